# Initial kernel scaffold; baseline (speedup 1.0000x reference)
import jax
import jax.numpy as jnp
from jax.experimental import pallas as pl


def kernel(x, edge_index, edge_attr, params):
    raise NotImplementedError("write your pallas kernel here")


# speedup vs baseline: 9.9827x; 9.9827x over previous
import functools

import jax
import jax.numpy as jnp
from jax import lax
from jax.experimental import pallas as pl
from jax.experimental.pallas import tpu as pltpu

N = 10000
E = 160000
H = 128
ED = 64
TOWERS = 4
LAYERS = 3
F_IN = H
F_OUT = H // TOWERS

EB = 2000
NB = 1000


def _msg_body(xi_ref, xj_ref, ea_ref, wee_ref, bee_ref, w0_ref, b0_ref,
              w1_ref, b1_ref, out_ref):
    ea128 = ea_ref[...] @ wee_ref[...] + bee_ref[...]
    h = jnp.concatenate([xi_ref[...], xj_ref[...], ea128], axis=1)
    h1 = jnp.maximum(h @ w0_ref[...] + b0_ref[...], 0.0)
    for t in range(TOWERS):
        s = t * F_IN
        out_ref[:, s:s + F_IN] = (
            h1[:, s:s + F_IN] @ w1_ref[t] + b1_ref[:, s:s + F_IN])


def _msg_call(xi, xj, ea, wee, bee, w0, b0, w1, b1):
    grid = (E // EB,)
    return pl.pallas_call(
        _msg_body,
        grid=grid,
        in_specs=[
            pl.BlockSpec((EB, F_IN), lambda i: (i, 0)),
            pl.BlockSpec((EB, F_IN), lambda i: (i, 0)),
            pl.BlockSpec((EB, ED), lambda i: (i, 0)),
            pl.BlockSpec((ED, F_IN), lambda i: (0, 0)),
            pl.BlockSpec((1, F_IN), lambda i: (0, 0)),
            pl.BlockSpec((3 * F_IN, TOWERS * F_IN), lambda i: (0, 0)),
            pl.BlockSpec((1, TOWERS * F_IN), lambda i: (0, 0)),
            pl.BlockSpec((TOWERS, F_IN, F_IN), lambda i: (0, 0, 0)),
            pl.BlockSpec((1, TOWERS * F_IN), lambda i: (0, 0)),
        ],
        out_specs=pl.BlockSpec((EB, TOWERS * F_IN), lambda i: (i, 0)),
        out_shape=jax.ShapeDtypeStruct((E, TOWERS * F_IN), jnp.float32),
    )(xi, xj, ea, wee, bee, w0, b0, w1, b1)


def _node_body(x_ref, sum_ref, sq_ref, mx_ref, rc_ref, he_ref,
               w0_ref, b0_ref, w1_ref, b1_ref, wl_ref, bl_ref,
               wih_ref, bih_ref, whh_ref, bhh_ref, out_ref):
    x = x_ref[...]
    rc = rc_ref[...]
    he = he_ref[...]
    mean = sum_ref[...] * rc
    m2 = sq_ref[...] * rc
    var = m2 - mean * mean
    std = jnp.sqrt(jnp.maximum(var, 0.0) + 1e-5)
    mx = jnp.where(he > 0.0, mx_ref[...], 0.0)
    outs = []
    for t in range(TOWERS):
        s = t * F_IN
        hin = jnp.concatenate(
            [x, mean[:, s:s + F_IN], mx[:, s:s + F_IN], std[:, s:s + F_IN]],
            axis=1)
        o = jnp.maximum(hin @ w0_ref[t] + b0_ref[:, t * F_OUT:(t + 1) * F_OUT],
                        0.0)
        outs.append(o @ w1_ref[t] + b1_ref[:, t * F_OUT:(t + 1) * F_OUT])
    mcat = jnp.concatenate(outs, axis=1)
    m = mcat @ wl_ref[...] + bl_ref[...]
    gi = m @ wih_ref[...] + bih_ref[...]
    gh = x @ whh_ref[...] + bhh_ref[...]
    r = jax.nn.sigmoid(gi[:, :H] + gh[:, :H])
    z = jax.nn.sigmoid(gi[:, H:2 * H] + gh[:, H:2 * H])
    n = jnp.tanh(gi[:, 2 * H:] + r * gh[:, 2 * H:])
    out_ref[...] = (1.0 - z) * n + z * x


def _node_call(x, ssum, ssq, smx, rc, he, w0, b0, w1, b1, wl, bl,
               wih, bih, whh, bhh):
    grid = (N // NB,)
    A = TOWERS * F_IN
    return pl.pallas_call(
        _node_body,
        grid=grid,
        in_specs=[
            pl.BlockSpec((NB, F_IN), lambda i: (i, 0)),
            pl.BlockSpec((NB, A), lambda i: (i, 0)),
            pl.BlockSpec((NB, A), lambda i: (i, 0)),
            pl.BlockSpec((NB, A), lambda i: (i, 0)),
            pl.BlockSpec((NB, 1), lambda i: (i, 0)),
            pl.BlockSpec((NB, 1), lambda i: (i, 0)),
            pl.BlockSpec((TOWERS, 4 * F_IN, F_OUT), lambda i: (0, 0, 0)),
            pl.BlockSpec((1, TOWERS * F_OUT), lambda i: (0, 0)),
            pl.BlockSpec((TOWERS, F_OUT, F_OUT), lambda i: (0, 0, 0)),
            pl.BlockSpec((1, TOWERS * F_OUT), lambda i: (0, 0)),
            pl.BlockSpec((H, H), lambda i: (0, 0)),
            pl.BlockSpec((1, H), lambda i: (0, 0)),
            pl.BlockSpec((H, 3 * H), lambda i: (0, 0)),
            pl.BlockSpec((1, 3 * H), lambda i: (0, 0)),
            pl.BlockSpec((H, 3 * H), lambda i: (0, 0)),
            pl.BlockSpec((1, 3 * H), lambda i: (0, 0)),
        ],
        out_specs=pl.BlockSpec((NB, H), lambda i: (i, 0)),
        out_shape=jax.ShapeDtypeStruct((N, H), jnp.float32),
    )(x, ssum, ssq, smx, rc, he, w0, b0, w1, b1, wl, bl, wih, bih, whh, bhh)


def _edge_body(u_ref, v_ref, ea_ref,
               au0_ref, cu0_ref, au1_ref, cu1_ref,
               av0_ref, cv0_ref, av1_ref, cv1_ref,
               f0_ref, g0_ref, f1_ref, g1_ref,
               e0_ref, d0_ref, e1_ref, d1_ref, out_ref):
    fu = jnp.maximum(u_ref[...] @ au0_ref[...] + cu0_ref[...], 0.0)
    fu = jnp.maximum(fu @ au1_ref[...] + cu1_ref[...], 0.0)
    fv = jnp.maximum(v_ref[...] @ av0_ref[...] + cv0_ref[...], 0.0)
    fv = jnp.maximum(fv @ av1_ref[...] + cv1_ref[...], 0.0)
    p = fu * fv
    m1 = jnp.maximum(p @ f0_ref[...] + g0_ref[...], 0.0)
    m_ed = jnp.maximum(m1 @ f1_ref[...] + g1_ref[...], 0.0)
    cat = jnp.concatenate([m_ed, ea_ref[...]], axis=1)
    h = jnp.maximum(cat @ e0_ref[...] + d0_ref[...], 0.0)
    out_ref[...] = h @ e1_ref[...] + d1_ref[...]


def _edge_call(u, v, ea, weights):
    grid = (E // EB,)
    specs = [
        pl.BlockSpec((EB, H), lambda i: (i, 0)),
        pl.BlockSpec((EB, H), lambda i: (i, 0)),
        pl.BlockSpec((EB, ED), lambda i: (i, 0)),
    ]
    for w in weights:
        specs.append(pl.BlockSpec(w.shape, lambda i, _r=len(w.shape): (0,) * _r))
    return pl.pallas_call(
        _edge_body,
        grid=grid,
        in_specs=specs,
        out_specs=pl.BlockSpec((EB, ED), lambda i: (i, 0)),
        out_shape=jax.ShapeDtypeStruct((E, ED), jnp.float32),
    )(u, v, ea, *weights)


def _prep_params(params):
    prepped = {"conv": []}
    for p in params["conv"]:
        q = {}
        q["wee"] = p["ee"][0].T
        q["bee"] = p["ee"][1][None, :]
        q["w0"] = jnp.concatenate([p["pre"][t][0][0].T for t in range(TOWERS)],
                                  axis=1)
        q["b0"] = jnp.concatenate([p["pre"][t][0][1] for t in range(TOWERS)]
                                  )[None, :]
        q["w1"] = jnp.stack([p["pre"][t][1][0].T for t in range(TOWERS)])
        q["b1"] = jnp.concatenate([p["pre"][t][1][1] for t in range(TOWERS)]
                                  )[None, :]
        q["pw0"] = jnp.stack([p["post"][t][0][0].T for t in range(TOWERS)])
        q["pb0"] = jnp.concatenate([p["post"][t][0][1] for t in range(TOWERS)]
                                   )[None, :]
        q["pw1"] = jnp.stack([p["post"][t][1][0].T for t in range(TOWERS)])
        q["pb1"] = jnp.concatenate([p["post"][t][1][1] for t in range(TOWERS)]
                                   )[None, :]
        q["wl"] = p["lin"][0].T
        q["bl"] = p["lin"][1][None, :]
        prepped["conv"].append(q)
    prepped["wih"] = params["gru_Wih"].T
    prepped["bih"] = params["gru_bih"][None, :]
    prepped["whh"] = params["gru_Whh"].T
    prepped["bhh"] = params["gru_bhh"][None, :]
    ew = []
    for key in ("fc_u", "fc_v", "fc", "eu"):
        for (W, b) in params[key]:
            ew.append(W.T)
            ew.append(b[None, :])
    prepped["edge_w"] = ew
    return prepped


def _gather_rows(table, idx):
    return jnp.take(table, idx, axis=0)


def _segment_reduce(msg, dst):
    ssum = jax.ops.segment_sum(msg, dst, num_segments=N)
    ssq = jax.ops.segment_sum(msg * msg, dst, num_segments=N)
    smx = jax.ops.segment_max(msg, dst, num_segments=N)
    return ssum, ssq, smx


def kernel(x, edge_index, edge_attr, params):
    src = edge_index[0]
    dst = edge_index[1]
    p = _prep_params(params)

    cnt = jax.ops.segment_sum(jnp.ones((E,), jnp.float32), dst,
                              num_segments=N)
    rc = (1.0 / jnp.maximum(cnt, 1.0))[:, None]
    he = (cnt > 0).astype(jnp.float32)[:, None]

    out = x
    ea = edge_attr
    for i in range(LAYERS):
        q = p["conv"][i]
        xi = _gather_rows(out, dst)
        xj = _gather_rows(out, src)
        msg = _msg_call(xi, xj, ea, q["wee"], q["bee"], q["w0"], q["b0"],
                        q["w1"], q["b1"])
        ssum, ssq, smx = _segment_reduce(msg, dst)
        out = _node_call(out, ssum, ssq, smx, rc, he,
                         q["pw0"], q["pb0"], q["pw1"], q["pb1"],
                         q["wl"], q["bl"],
                         p["wih"], p["bih"], p["whh"], p["bhh"])
        u = _gather_rows(out, src)
        v = _gather_rows(out, dst)
        ea = _edge_call(u, v, ea, p["edge_w"])
    return (out, ea, out)


# speedup vs baseline: 25.7286x; 2.5773x over previous
import functools

import jax
import jax.numpy as jnp
from jax import lax
from jax.experimental import pallas as pl
from jax.experimental.pallas import tpu as pltpu
from jax.experimental.pallas import tpu_sc as plsc

N = 10000
E = 160000
H = 128
ED = 64
TOWERS = 4
LAYERS = 3
F_IN = H
F_OUT = H // TOWERS

EB = 2000
NB = 1000


def _msg_body(xi_ref, xj_ref, ea_ref, wee_ref, bee_ref, w0_ref, b0_ref,
              w1_ref, b1_ref, out_ref):
    ea128 = ea_ref[...] @ wee_ref[...] + bee_ref[...]
    h = jnp.concatenate([xi_ref[...], xj_ref[...], ea128], axis=1)
    h1 = jnp.maximum(h @ w0_ref[...] + b0_ref[...], 0.0)
    for t in range(TOWERS):
        s = t * F_IN
        out_ref[:, s:s + F_IN] = (
            h1[:, s:s + F_IN] @ w1_ref[t] + b1_ref[:, s:s + F_IN])


def _msg_call(xi, xj, ea, wee, bee, w0, b0, w1, b1):
    grid = (E // EB,)
    return pl.pallas_call(
        _msg_body,
        grid=grid,
        in_specs=[
            pl.BlockSpec((EB, F_IN), lambda i: (i, 0)),
            pl.BlockSpec((EB, F_IN), lambda i: (i, 0)),
            pl.BlockSpec((EB, ED), lambda i: (i, 0)),
            pl.BlockSpec((ED, F_IN), lambda i: (0, 0)),
            pl.BlockSpec((1, F_IN), lambda i: (0, 0)),
            pl.BlockSpec((3 * F_IN, TOWERS * F_IN), lambda i: (0, 0)),
            pl.BlockSpec((1, TOWERS * F_IN), lambda i: (0, 0)),
            pl.BlockSpec((TOWERS, F_IN, F_IN), lambda i: (0, 0, 0)),
            pl.BlockSpec((1, TOWERS * F_IN), lambda i: (0, 0)),
        ],
        out_specs=pl.BlockSpec((EB, TOWERS * F_IN), lambda i: (i, 0)),
        out_shape=jax.ShapeDtypeStruct((E_PAD, TOWERS * F_IN), jnp.float32),
    )(xi, xj, ea, wee, bee, w0, b0, w1, b1)


def _node_body(x_ref, sum_ref, sq_ref, mx_ref, rc_ref, he_ref,
               w0_ref, b0_ref, w1_ref, b1_ref, wl_ref, bl_ref,
               wih_ref, bih_ref, whh_ref, bhh_ref, out_ref):
    x = x_ref[...]
    rc = rc_ref[...]
    he = he_ref[...]
    mean = sum_ref[...] * rc
    m2 = sq_ref[...] * rc
    var = m2 - mean * mean
    std = jnp.sqrt(jnp.maximum(var, 0.0) + 1e-5)
    mx = jnp.where(he > 0.0, mx_ref[...], 0.0)
    outs = []
    for t in range(TOWERS):
        s = t * F_IN
        hin = jnp.concatenate(
            [x, mean[:, s:s + F_IN], mx[:, s:s + F_IN], std[:, s:s + F_IN]],
            axis=1)
        o = jnp.maximum(hin @ w0_ref[t] + b0_ref[:, t * F_OUT:(t + 1) * F_OUT],
                        0.0)
        outs.append(o @ w1_ref[t] + b1_ref[:, t * F_OUT:(t + 1) * F_OUT])
    mcat = jnp.concatenate(outs, axis=1)
    m = mcat @ wl_ref[...] + bl_ref[...]
    gi = m @ wih_ref[...] + bih_ref[...]
    gh = x @ whh_ref[...] + bhh_ref[...]
    r = jax.nn.sigmoid(gi[:, :H] + gh[:, :H])
    z = jax.nn.sigmoid(gi[:, H:2 * H] + gh[:, H:2 * H])
    n = jnp.tanh(gi[:, 2 * H:] + r * gh[:, 2 * H:])
    out_ref[...] = (1.0 - z) * n + z * x


def _node_call(x, ssum, ssq, smx, rc, he, w0, b0, w1, b1, wl, bl,
               wih, bih, whh, bhh):
    grid = (N // NB,)
    A = TOWERS * F_IN
    return pl.pallas_call(
        _node_body,
        grid=grid,
        in_specs=[
            pl.BlockSpec((NB, F_IN), lambda i: (i, 0)),
            pl.BlockSpec((NB, A), lambda i: (i, 0)),
            pl.BlockSpec((NB, A), lambda i: (i, 0)),
            pl.BlockSpec((NB, A), lambda i: (i, 0)),
            pl.BlockSpec((NB, 1), lambda i: (i, 0)),
            pl.BlockSpec((NB, 1), lambda i: (i, 0)),
            pl.BlockSpec((TOWERS, 4 * F_IN, F_OUT), lambda i: (0, 0, 0)),
            pl.BlockSpec((1, TOWERS * F_OUT), lambda i: (0, 0)),
            pl.BlockSpec((TOWERS, F_OUT, F_OUT), lambda i: (0, 0, 0)),
            pl.BlockSpec((1, TOWERS * F_OUT), lambda i: (0, 0)),
            pl.BlockSpec((H, H), lambda i: (0, 0)),
            pl.BlockSpec((1, H), lambda i: (0, 0)),
            pl.BlockSpec((H, 3 * H), lambda i: (0, 0)),
            pl.BlockSpec((1, 3 * H), lambda i: (0, 0)),
            pl.BlockSpec((H, 3 * H), lambda i: (0, 0)),
            pl.BlockSpec((1, 3 * H), lambda i: (0, 0)),
        ],
        out_specs=pl.BlockSpec((NB, H), lambda i: (i, 0)),
        out_shape=jax.ShapeDtypeStruct((N, H), jnp.float32),
    )(x, ssum, ssq, smx, rc, he, w0, b0, w1, b1, wl, bl, wih, bih, whh, bhh)


def _edge_body(u_ref, v_ref, ea_ref,
               au0_ref, cu0_ref, au1_ref, cu1_ref,
               av0_ref, cv0_ref, av1_ref, cv1_ref,
               f0_ref, g0_ref, f1_ref, g1_ref,
               e0_ref, d0_ref, e1_ref, d1_ref, out_ref):
    fu = jnp.maximum(u_ref[...] @ au0_ref[...] + cu0_ref[...], 0.0)
    fu = jnp.maximum(fu @ au1_ref[...] + cu1_ref[...], 0.0)
    fv = jnp.maximum(v_ref[...] @ av0_ref[...] + cv0_ref[...], 0.0)
    fv = jnp.maximum(fv @ av1_ref[...] + cv1_ref[...], 0.0)
    p = fu * fv
    m1 = jnp.maximum(p @ f0_ref[...] + g0_ref[...], 0.0)
    m_ed = jnp.maximum(m1 @ f1_ref[...] + g1_ref[...], 0.0)
    cat = jnp.concatenate([m_ed, ea_ref[...]], axis=1)
    h = jnp.maximum(cat @ e0_ref[...] + d0_ref[...], 0.0)
    out_ref[...] = h @ e1_ref[...] + d1_ref[...]


def _edge_call(u, v, ea, weights):
    grid = (E // EB,)
    specs = [
        pl.BlockSpec((EB, H), lambda i: (i, 0)),
        pl.BlockSpec((EB, H), lambda i: (i, 0)),
        pl.BlockSpec((EB, ED), lambda i: (i, 0)),
    ]
    for w in weights:
        specs.append(pl.BlockSpec(w.shape, lambda i, _r=len(w.shape): (0,) * _r))
    return pl.pallas_call(
        _edge_body,
        grid=grid,
        in_specs=specs,
        out_specs=pl.BlockSpec((EB, ED), lambda i: (i, 0)),
        out_shape=jax.ShapeDtypeStruct((E, ED), jnp.float32),
    )(u, v, ea, *weights)


def _prep_params(params):
    prepped = {"conv": []}
    for p in params["conv"]:
        q = {}
        q["wee"] = p["ee"][0].T
        q["bee"] = p["ee"][1][None, :]
        q["w0"] = jnp.concatenate([p["pre"][t][0][0].T for t in range(TOWERS)],
                                  axis=1)
        q["b0"] = jnp.concatenate([p["pre"][t][0][1] for t in range(TOWERS)]
                                  )[None, :]
        q["w1"] = jnp.stack([p["pre"][t][1][0].T for t in range(TOWERS)])
        q["b1"] = jnp.concatenate([p["pre"][t][1][1] for t in range(TOWERS)]
                                  )[None, :]
        q["pw0"] = jnp.stack([p["post"][t][0][0].T for t in range(TOWERS)])
        q["pb0"] = jnp.concatenate([p["post"][t][0][1] for t in range(TOWERS)]
                                   )[None, :]
        q["pw1"] = jnp.stack([p["post"][t][1][0].T for t in range(TOWERS)])
        q["pb1"] = jnp.concatenate([p["post"][t][1][1] for t in range(TOWERS)]
                                   )[None, :]
        q["wl"] = p["lin"][0].T
        q["bl"] = p["lin"][1][None, :]
        prepped["conv"].append(q)
    prepped["wih"] = params["gru_Wih"].T
    prepped["bih"] = params["gru_bih"][None, :]
    prepped["whh"] = params["gru_Whh"].T
    prepped["bhh"] = params["gru_bhh"][None, :]
    ew = []
    for key in ("fc_u", "fc_v", "fc", "eu"):
        for (W, b) in params[key]:
            ew.append(W.T)
            ew.append(b[None, :])
    prepped["edge_w"] = ew
    return prepped


NW = 32
GW = 128
CH = 32
NPT = 320
NPAD = NW * NPT
E_PAD = E + CH

_MESH = plsc.VectorSubcoreMesh(core_axis_name="c", subcore_axis_name="s")


def _sc_gather_pair(table, idx_a, idx_b):
    D = table.shape[1]

    @functools.partial(
        pl.kernel,
        out_type=[jax.ShapeDtypeStruct((E, D), jnp.float32),
                  jax.ShapeDtypeStruct((E, D), jnp.float32)],
        mesh=_MESH,
    )
    def k(tab_hbm, ia_hbm, ib_hbm, oa_hbm, ob_hbm):
        def body(ia_vmem, ib_vmem, oa_vmem, ob_vmem):
            pltpu.sync_copy(tab_hbm.at[ia_vmem.at[0]], oa_vmem)
            pltpu.sync_copy(tab_hbm.at[ib_vmem.at[0]], ob_vmem)

        pltpu.emit_pipeline(
            body,
            grid=(E // GW,),
            in_specs=[
                pl.BlockSpec((1, GW), lambda i: (0, i)),
                pl.BlockSpec((1, GW), lambda i: (0, i)),
            ],
            out_specs=[
                pl.BlockSpec((GW, D), lambda i: (i, 0)),
                pl.BlockSpec((GW, D), lambda i: (i, 0)),
            ],
            core_axis_name=("c", "s"),
            dimension_semantics=(pltpu.PARALLEL,),
        )(ia_hbm, ib_hbm, oa_hbm, ob_hbm)

    return k(table, idx_a, idx_b)


def _sc_gather_one(table, idx):
    D = table.shape[1]

    @functools.partial(
        pl.kernel,
        out_type=jax.ShapeDtypeStruct((E, D), jnp.float32),
        mesh=_MESH,
        compiler_params=pltpu.CompilerParams(use_tc_tiling_on_sc=False),
    )
    def k(tab_hbm, i_hbm, o_hbm):
        def body(i_vmem, o_vmem):
            pltpu.sync_copy(tab_hbm.at[i_vmem.at[0]], o_vmem)

        pltpu.emit_pipeline(
            body,
            grid=(E // GW,),
            in_specs=[pl.BlockSpec((1, GW), lambda i: (0, i))],
            out_specs=[pl.BlockSpec((GW, D), lambda i: (i, 0))],
            core_axis_name=("c", "s"),
            dimension_semantics=(pltpu.PARALLEL,),
        )(i_hbm, o_hbm)

    return k(table, idx)


def _sc_segreduce(msg_padded, off_padded):
    A = TOWERS * F_IN
    G = 8

    @functools.partial(
        pl.kernel,
        out_type=[jax.ShapeDtypeStruct((N, A), jnp.float32)] * 3,
        mesh=_MESH,
        scratch_types=[
            pltpu.VMEM((NPT + 16,), jnp.int32),
            pltpu.VMEM((CH, A), jnp.float32),
            pltpu.VMEM((G, A), jnp.float32),
            pltpu.VMEM((G, A), jnp.float32),
            pltpu.VMEM((G, A), jnp.float32),
        ],
    )
    def k(msg_hbm, off_hbm, sum_hbm, sq_hbm, mx_hbm,
          off_v, buf, st_sum, st_sq, st_mx):
        wid = lax.axis_index("s") * 2 + lax.axis_index("c")
        n0 = pl.multiple_of(wid * NPT, 8)
        pltpu.sync_copy(off_hbm.at[pl.ds(n0, NPT + 16)], off_v)
        nn = jnp.minimum(NPT, N - n0)
        ng = nn // G

        @pl.loop(0, ng)
        def g_body(g):
            @pl.loop(0, G)
            def n_body(kk):
                i = g * G + kk
                ovec = off_v[pl.ds(i, 16)]
                s = ovec[0]
                e = ovec[1]
                zero = jnp.zeros((16,), jnp.float32)
                ninf = jnp.full((16,), -3.4e38, jnp.float32)
                for f in range(A // 16):
                    st_sum[kk, pl.ds(f * 16, 16)] = zero
                    st_sq[kk, pl.ds(f * 16, 16)] = zero
                    st_mx[kk, pl.ds(f * 16, 16)] = ninf
                s_al = (s // 8) * 8
                nch = (e - s_al + (CH - 1)) // CH

                @pl.loop(0, nch)
                def c_body(c, _kk=kk, _s=s, _e=e, _s_al=s_al):
                    base = pl.multiple_of(_s_al + c * CH, 8)
                    pltpu.sync_copy(msg_hbm.at[pl.ds(base, CH)], buf)
                    lo = jnp.maximum(_s - base, 0)
                    valid = jnp.minimum(CH, _e - base)
                    for fq in range(4):
                        def e_body(j, carry, _fq=fq):
                            acc = list(carry)
                            for t in range(8):
                                v = buf[j, pl.ds(_fq * 128 + t * 16, 16)]
                                acc[t] = acc[t] + v
                                acc[8 + t] = acc[8 + t] + v * v
                                acc[16 + t] = jnp.maximum(acc[16 + t], v)
                            return tuple(acc)

                        init = tuple(
                            [st_sum[_kk, pl.ds(fq * 128 + t * 16, 16)]
                             for t in range(8)]
                            + [st_sq[_kk, pl.ds(fq * 128 + t * 16, 16)]
                               for t in range(8)]
                            + [st_mx[_kk, pl.ds(fq * 128 + t * 16, 16)]
                               for t in range(8)])
                        res = lax.fori_loop(lo, valid, e_body, init)
                        for t in range(8):
                            st_sum[_kk, pl.ds(fq * 128 + t * 16, 16)] = res[t]
                            st_sq[_kk, pl.ds(fq * 128 + t * 16, 16)] = res[8 + t]
                            st_mx[_kk, pl.ds(fq * 128 + t * 16, 16)] = res[16 + t]

            row0 = pl.multiple_of(n0 + g * G, 8)
            pltpu.sync_copy(st_sum, sum_hbm.at[pl.ds(row0, G)])
            pltpu.sync_copy(st_sq, sq_hbm.at[pl.ds(row0, G)])
            pltpu.sync_copy(st_mx, mx_hbm.at[pl.ds(row0, G)])

    return k(msg_padded, off_padded)


def kernel(x, edge_index, edge_attr, params):
    src = edge_index[0]
    dst = edge_index[1]
    p = _prep_params(params)

    eid = jnp.arange(E, dtype=jnp.int32)
    sdst, ssrc, perm = lax.sort((dst, src, eid), num_keys=1)
    off = jnp.searchsorted(sdst, jnp.arange(N + 1, dtype=jnp.int32)
                           ).astype(jnp.int32)
    off_padded = jnp.concatenate(
        [off, jnp.full((NPAD + 16 - (N + 1),), E, jnp.int32)])
    cnt = (off[1:] - off[:-1]).astype(jnp.float32)
    rc = (1.0 / jnp.maximum(cnt, 1.0))[:, None]
    he = (cnt > 0).astype(jnp.float32)[:, None]
    invp = jnp.zeros((E,), jnp.int32).at[perm].set(eid)

    sdst2d = sdst.reshape(1, E)
    ssrc2d = ssrc.reshape(1, E)
    ea = _sc_gather_one(edge_attr, perm.reshape(1, E))
    out = x
    for i in range(LAYERS):
        q = p["conv"][i]
        xi, xj = _sc_gather_pair(out, sdst2d, ssrc2d)
        msg = _msg_call(xi, xj, ea, q["wee"], q["bee"], q["w0"], q["b0"],
                        q["w1"], q["b1"])
        ssum, ssq, smx = _sc_segreduce(msg, off_padded)
        out = _node_call(out, ssum, ssq, smx, rc, he,
                         q["pw0"], q["pb0"], q["pw1"], q["pb1"],
                         q["wl"], q["bl"],
                         p["wih"], p["bih"], p["whh"], p["bhh"])
        u, v = _sc_gather_pair(out, ssrc2d, sdst2d)
        ea = _edge_call(u, v, ea, p["edge_w"])
    ea = _sc_gather_one(ea, invp.reshape(1, E))
    return (out, ea, out)


# speedup vs baseline: 27.7442x; 1.0783x over previous
import functools

import jax
import jax.numpy as jnp
from jax import lax
from jax.experimental import pallas as pl
from jax.experimental.pallas import tpu as pltpu
from jax.experimental.pallas import tpu_sc as plsc

N = 10000
E = 160000
H = 128
ED = 64
TOWERS = 4
LAYERS = 3
F_IN = H
F_OUT = H // TOWERS

EB = 2000
NB = 1000


def _msg_body(xi_ref, xj_ref, ea_ref, wee_ref, bee_ref, w0_ref, b0_ref,
              w1_ref, b1_ref, out_ref):
    ea128 = ea_ref[...] @ wee_ref[...] + bee_ref[...]
    h = jnp.concatenate([xi_ref[...], xj_ref[...], ea128], axis=1)
    h1 = jnp.maximum(h @ w0_ref[...] + b0_ref[...], 0.0)
    for t in range(TOWERS):
        s = t * F_IN
        out_ref[:, s:s + F_IN] = (
            h1[:, s:s + F_IN] @ w1_ref[t] + b1_ref[:, s:s + F_IN])


def _msg_call(xi, xj, ea, wee, bee, w0, b0, w1, b1):
    grid = (E // EB,)
    return pl.pallas_call(
        _msg_body,
        grid=grid,
        in_specs=[
            pl.BlockSpec((EB, F_IN), lambda i: (i, 0)),
            pl.BlockSpec((EB, F_IN), lambda i: (i, 0)),
            pl.BlockSpec((EB, ED), lambda i: (i, 0)),
            pl.BlockSpec((ED, F_IN), lambda i: (0, 0)),
            pl.BlockSpec((1, F_IN), lambda i: (0, 0)),
            pl.BlockSpec((3 * F_IN, TOWERS * F_IN), lambda i: (0, 0)),
            pl.BlockSpec((1, TOWERS * F_IN), lambda i: (0, 0)),
            pl.BlockSpec((TOWERS, F_IN, F_IN), lambda i: (0, 0, 0)),
            pl.BlockSpec((1, TOWERS * F_IN), lambda i: (0, 0)),
        ],
        out_specs=pl.BlockSpec((EB, TOWERS * F_IN), lambda i: (i, 0)),
        out_shape=jax.ShapeDtypeStruct((E_PAD, TOWERS * F_IN), jnp.float32),
    )(xi, xj, ea, wee, bee, w0, b0, w1, b1)


def _node_body(x_ref, sum_ref, sq_ref, mx_ref, rc_ref, he_ref,
               w0_ref, b0_ref, w1_ref, b1_ref, wl_ref, bl_ref,
               wih_ref, bih_ref, whh_ref, bhh_ref, out_ref):
    x = x_ref[...]
    rc = rc_ref[...]
    he = he_ref[...]
    mean = sum_ref[...] * rc
    m2 = sq_ref[...] * rc
    var = m2 - mean * mean
    std = jnp.sqrt(jnp.maximum(var, 0.0) + 1e-5)
    mx = jnp.where(he > 0.0, mx_ref[...], 0.0)
    outs = []
    for t in range(TOWERS):
        s = t * F_IN
        hin = jnp.concatenate(
            [x, mean[:, s:s + F_IN], mx[:, s:s + F_IN], std[:, s:s + F_IN]],
            axis=1)
        o = jnp.maximum(hin @ w0_ref[t] + b0_ref[:, t * F_OUT:(t + 1) * F_OUT],
                        0.0)
        outs.append(o @ w1_ref[t] + b1_ref[:, t * F_OUT:(t + 1) * F_OUT])
    mcat = jnp.concatenate(outs, axis=1)
    m = mcat @ wl_ref[...] + bl_ref[...]
    gi = m @ wih_ref[...] + bih_ref[...]
    gh = x @ whh_ref[...] + bhh_ref[...]
    r = jax.nn.sigmoid(gi[:, :H] + gh[:, :H])
    z = jax.nn.sigmoid(gi[:, H:2 * H] + gh[:, H:2 * H])
    n = jnp.tanh(gi[:, 2 * H:] + r * gh[:, 2 * H:])
    out_ref[...] = (1.0 - z) * n + z * x


def _node_call(x, ssum, ssq, smx, rc, he, w0, b0, w1, b1, wl, bl,
               wih, bih, whh, bhh):
    grid = (N // NB,)
    A = TOWERS * F_IN
    return pl.pallas_call(
        _node_body,
        grid=grid,
        in_specs=[
            pl.BlockSpec((NB, F_IN), lambda i: (i, 0)),
            pl.BlockSpec((NB, A), lambda i: (i, 0)),
            pl.BlockSpec((NB, A), lambda i: (i, 0)),
            pl.BlockSpec((NB, A), lambda i: (i, 0)),
            pl.BlockSpec((NB, 1), lambda i: (i, 0)),
            pl.BlockSpec((NB, 1), lambda i: (i, 0)),
            pl.BlockSpec((TOWERS, 4 * F_IN, F_OUT), lambda i: (0, 0, 0)),
            pl.BlockSpec((1, TOWERS * F_OUT), lambda i: (0, 0)),
            pl.BlockSpec((TOWERS, F_OUT, F_OUT), lambda i: (0, 0, 0)),
            pl.BlockSpec((1, TOWERS * F_OUT), lambda i: (0, 0)),
            pl.BlockSpec((H, H), lambda i: (0, 0)),
            pl.BlockSpec((1, H), lambda i: (0, 0)),
            pl.BlockSpec((H, 3 * H), lambda i: (0, 0)),
            pl.BlockSpec((1, 3 * H), lambda i: (0, 0)),
            pl.BlockSpec((H, 3 * H), lambda i: (0, 0)),
            pl.BlockSpec((1, 3 * H), lambda i: (0, 0)),
        ],
        out_specs=pl.BlockSpec((NB, H), lambda i: (i, 0)),
        out_shape=jax.ShapeDtypeStruct((N, H), jnp.float32),
    )(x, ssum, ssq, smx, rc, he, w0, b0, w1, b1, wl, bl, wih, bih, whh, bhh)


def _edge_body(u_ref, v_ref, ea_ref,
               au0_ref, cu0_ref, au1_ref, cu1_ref,
               av0_ref, cv0_ref, av1_ref, cv1_ref,
               f0_ref, g0_ref, f1_ref, g1_ref,
               e0_ref, d0_ref, e1_ref, d1_ref, out_ref):
    fu = jnp.maximum(u_ref[...] @ au0_ref[...] + cu0_ref[...], 0.0)
    fu = jnp.maximum(fu @ au1_ref[...] + cu1_ref[...], 0.0)
    fv = jnp.maximum(v_ref[...] @ av0_ref[...] + cv0_ref[...], 0.0)
    fv = jnp.maximum(fv @ av1_ref[...] + cv1_ref[...], 0.0)
    p = fu * fv
    m1 = jnp.maximum(p @ f0_ref[...] + g0_ref[...], 0.0)
    m_ed = jnp.maximum(m1 @ f1_ref[...] + g1_ref[...], 0.0)
    cat = jnp.concatenate([m_ed, ea_ref[...]], axis=1)
    h = jnp.maximum(cat @ e0_ref[...] + d0_ref[...], 0.0)
    out_ref[...] = h @ e1_ref[...] + d1_ref[...]


def _edge_call(u, v, ea, weights):
    grid = (E // EB,)
    specs = [
        pl.BlockSpec((EB, H), lambda i: (i, 0)),
        pl.BlockSpec((EB, H), lambda i: (i, 0)),
        pl.BlockSpec((EB, ED), lambda i: (i, 0)),
    ]
    for w in weights:
        specs.append(pl.BlockSpec(w.shape, lambda i, _r=len(w.shape): (0,) * _r))
    return pl.pallas_call(
        _edge_body,
        grid=grid,
        in_specs=specs,
        out_specs=pl.BlockSpec((EB, ED), lambda i: (i, 0)),
        out_shape=jax.ShapeDtypeStruct((E, ED), jnp.float32),
    )(u, v, ea, *weights)


def _prep_params(params):
    prepped = {"conv": []}
    for p in params["conv"]:
        q = {}
        q["wee"] = p["ee"][0].T
        q["bee"] = p["ee"][1][None, :]
        q["w0"] = jnp.concatenate([p["pre"][t][0][0].T for t in range(TOWERS)],
                                  axis=1)
        q["b0"] = jnp.concatenate([p["pre"][t][0][1] for t in range(TOWERS)]
                                  )[None, :]
        q["w1"] = jnp.stack([p["pre"][t][1][0].T for t in range(TOWERS)])
        q["b1"] = jnp.concatenate([p["pre"][t][1][1] for t in range(TOWERS)]
                                  )[None, :]
        q["pw0"] = jnp.stack([p["post"][t][0][0].T for t in range(TOWERS)])
        q["pb0"] = jnp.concatenate([p["post"][t][0][1] for t in range(TOWERS)]
                                   )[None, :]
        q["pw1"] = jnp.stack([p["post"][t][1][0].T for t in range(TOWERS)])
        q["pb1"] = jnp.concatenate([p["post"][t][1][1] for t in range(TOWERS)]
                                   )[None, :]
        q["wl"] = p["lin"][0].T
        q["bl"] = p["lin"][1][None, :]
        prepped["conv"].append(q)
    prepped["wih"] = params["gru_Wih"].T
    prepped["bih"] = params["gru_bih"][None, :]
    prepped["whh"] = params["gru_Whh"].T
    prepped["bhh"] = params["gru_bhh"][None, :]
    ew = []
    for key in ("fc_u", "fc_v", "fc", "eu"):
        for (W, b) in params[key]:
            ew.append(W.T)
            ew.append(b[None, :])
    prepped["edge_w"] = ew
    return prepped


NW = 32
GW = 128
CH = 32
NPT = 320
NPAD = NW * NPT
E_PAD = E + CH

_MESH = plsc.VectorSubcoreMesh(core_axis_name="c", subcore_axis_name="s")


def _sc_gather_pair(table, idx_a, idx_b):
    D = table.shape[1]

    @functools.partial(
        pl.kernel,
        out_type=[jax.ShapeDtypeStruct((E, D), jnp.float32),
                  jax.ShapeDtypeStruct((E, D), jnp.float32)],
        mesh=_MESH,
        scratch_types=[pltpu.SemaphoreType.DMA, pltpu.SemaphoreType.DMA],
    )
    def k(tab_hbm, ia_hbm, ib_hbm, oa_hbm, ob_hbm, sema, semb):
        def body(ia_vmem, ib_vmem, oa_vmem, ob_vmem):
            ca = pltpu.async_copy(tab_hbm.at[ia_vmem.at[0]], oa_vmem, sema)
            cb = pltpu.async_copy(tab_hbm.at[ib_vmem.at[0]], ob_vmem, semb)
            ca.wait()
            cb.wait()

        pltpu.emit_pipeline(
            body,
            grid=(E // GW,),
            in_specs=[
                pl.BlockSpec((1, GW), lambda i: (0, i)),
                pl.BlockSpec((1, GW), lambda i: (0, i)),
            ],
            out_specs=[
                pl.BlockSpec((GW, D), lambda i: (i, 0)),
                pl.BlockSpec((GW, D), lambda i: (i, 0)),
            ],
            core_axis_name=("c", "s"),
            dimension_semantics=(pltpu.PARALLEL,),
        )(ia_hbm, ib_hbm, oa_hbm, ob_hbm)

    return k(table, idx_a, idx_b)


def _sc_gather_one(table, idx):
    D = table.shape[1]

    @functools.partial(
        pl.kernel,
        out_type=jax.ShapeDtypeStruct((E, D), jnp.float32),
        mesh=_MESH,
        compiler_params=pltpu.CompilerParams(use_tc_tiling_on_sc=False),
    )
    def k(tab_hbm, i_hbm, o_hbm):
        def body(i_vmem, o_vmem):
            pltpu.sync_copy(tab_hbm.at[i_vmem.at[0]], o_vmem)

        pltpu.emit_pipeline(
            body,
            grid=(E // GW,),
            in_specs=[pl.BlockSpec((1, GW), lambda i: (0, i))],
            out_specs=[pl.BlockSpec((GW, D), lambda i: (i, 0))],
            core_axis_name=("c", "s"),
            dimension_semantics=(pltpu.PARALLEL,),
        )(i_hbm, o_hbm)

    return k(table, idx)


def _sc_segreduce(msg_padded, off_padded):
    A = TOWERS * F_IN
    G = 8

    @functools.partial(
        pl.kernel,
        out_type=[jax.ShapeDtypeStruct((N, A), jnp.float32)] * 3,
        mesh=_MESH,
        scratch_types=[
            pltpu.VMEM((NPT + 16,), jnp.int32),
            pltpu.VMEM((CH, A), jnp.float32),
            pltpu.VMEM((G, A), jnp.float32),
            pltpu.VMEM((G, A), jnp.float32),
            pltpu.VMEM((G, A), jnp.float32),
        ],
    )
    def k(msg_hbm, off_hbm, sum_hbm, sq_hbm, mx_hbm,
          off_v, buf, st_sum, st_sq, st_mx):
        wid = lax.axis_index("s") * 2 + lax.axis_index("c")
        n0 = pl.multiple_of(wid * NPT, 8)
        pltpu.sync_copy(off_hbm.at[pl.ds(n0, NPT + 16)], off_v)
        nn = jnp.minimum(NPT, N - n0)
        ng = nn // G

        @pl.loop(0, ng)
        def g_body(g):
            @pl.loop(0, G)
            def n_body(kk):
                i = g * G + kk
                ovec = off_v[pl.ds(i, 16)]
                s = ovec[0]
                e = ovec[1]
                zero = jnp.zeros((16,), jnp.float32)
                ninf = jnp.full((16,), -3.4e38, jnp.float32)
                for f in range(A // 16):
                    st_sum[kk, pl.ds(f * 16, 16)] = zero
                    st_sq[kk, pl.ds(f * 16, 16)] = zero
                    st_mx[kk, pl.ds(f * 16, 16)] = ninf
                s_al = (s // 8) * 8
                nch = (e - s_al + (CH - 1)) // CH

                @pl.loop(0, nch)
                def c_body(c, _kk=kk, _s=s, _e=e, _s_al=s_al):
                    base = pl.multiple_of(_s_al + c * CH, 8)
                    pltpu.sync_copy(msg_hbm.at[pl.ds(base, CH)], buf)
                    lo = jnp.maximum(_s - base, 0)
                    valid = jnp.minimum(CH, _e - base)
                    for fq in range(4):
                        def e_body(j, carry, _fq=fq):
                            acc = list(carry)
                            for t in range(8):
                                v = buf[j, pl.ds(_fq * 128 + t * 16, 16)]
                                acc[t] = acc[t] + v
                                acc[8 + t] = acc[8 + t] + v * v
                                acc[16 + t] = jnp.maximum(acc[16 + t], v)
                            return tuple(acc)

                        init = tuple(
                            [st_sum[_kk, pl.ds(fq * 128 + t * 16, 16)]
                             for t in range(8)]
                            + [st_sq[_kk, pl.ds(fq * 128 + t * 16, 16)]
                               for t in range(8)]
                            + [st_mx[_kk, pl.ds(fq * 128 + t * 16, 16)]
                               for t in range(8)])
                        res = lax.fori_loop(lo, valid, e_body, init)
                        for t in range(8):
                            st_sum[_kk, pl.ds(fq * 128 + t * 16, 16)] = res[t]
                            st_sq[_kk, pl.ds(fq * 128 + t * 16, 16)] = res[8 + t]
                            st_mx[_kk, pl.ds(fq * 128 + t * 16, 16)] = res[16 + t]

            row0 = pl.multiple_of(n0 + g * G, 8)
            pltpu.sync_copy(st_sum, sum_hbm.at[pl.ds(row0, G)])
            pltpu.sync_copy(st_sq, sq_hbm.at[pl.ds(row0, G)])
            pltpu.sync_copy(st_mx, mx_hbm.at[pl.ds(row0, G)])

    return k(msg_padded, off_padded)


def kernel(x, edge_index, edge_attr, params):
    src = edge_index[0]
    dst = edge_index[1]
    p = _prep_params(params)

    eid = jnp.arange(E, dtype=jnp.int32)
    sdst, ssrc, perm = lax.sort((dst, src, eid), num_keys=1)
    off = jnp.searchsorted(sdst, jnp.arange(N + 1, dtype=jnp.int32)
                           ).astype(jnp.int32)
    off_padded = jnp.concatenate(
        [off, jnp.full((NPAD + 16 - (N + 1),), E, jnp.int32)])
    cnt = (off[1:] - off[:-1]).astype(jnp.float32)
    rc = (1.0 / jnp.maximum(cnt, 1.0))[:, None]
    he = (cnt > 0).astype(jnp.float32)[:, None]
    invp = jnp.zeros((E,), jnp.int32).at[perm].set(eid)

    sdst2d = sdst.reshape(1, E)
    ssrc2d = ssrc.reshape(1, E)
    ea = _sc_gather_one(edge_attr, perm.reshape(1, E))
    out = x
    u = v = None
    for i in range(LAYERS):
        q = p["conv"][i]
        if i == 0:
            xi, xj = _sc_gather_pair(out, sdst2d, ssrc2d)
        else:
            xi, xj = v, u
        msg = _msg_call(xi, xj, ea, q["wee"], q["bee"], q["w0"], q["b0"],
                        q["w1"], q["b1"])
        ssum, ssq, smx = _sc_segreduce(msg, off_padded)
        out = _node_call(out, ssum, ssq, smx, rc, he,
                         q["pw0"], q["pb0"], q["pw1"], q["pb1"],
                         q["wl"], q["bl"],
                         p["wih"], p["bih"], p["whh"], p["bhh"])
        u, v = _sc_gather_pair(out, ssrc2d, sdst2d)
        ea = _edge_call(u, v, ea, p["edge_w"])
    ea = _sc_gather_one(ea, invp.reshape(1, E))
    return (out, ea, out)


# speedup vs baseline: 28.0023x; 1.0093x over previous
import functools

import jax
import jax.numpy as jnp
from jax import lax
from jax.experimental import pallas as pl
from jax.experimental.pallas import tpu as pltpu
from jax.experimental.pallas import tpu_sc as plsc

N = 10000
E = 160000
H = 128
ED = 64
TOWERS = 4
LAYERS = 3
F_IN = H
F_OUT = H // TOWERS

EB = 2000
NB = 1000


def _msg_body(xi_ref, xj_ref, ea_ref, wee_ref, bee_ref, w0_ref, b0_ref,
              w1_ref, b1_ref, out_ref):
    ea128 = ea_ref[...] @ wee_ref[...] + bee_ref[...]
    h = jnp.concatenate([xi_ref[...], xj_ref[...], ea128], axis=1)
    h1 = jnp.maximum(h @ w0_ref[...] + b0_ref[...], 0.0)
    for t in range(TOWERS):
        s = t * F_IN
        out_ref[:, s:s + F_IN] = (
            h1[:, s:s + F_IN] @ w1_ref[t] + b1_ref[:, s:s + F_IN])


def _msg_call(xi, xj, ea, wee, bee, w0, b0, w1, b1):
    grid = (E // EB,)
    return pl.pallas_call(
        _msg_body,
        grid=grid,
        in_specs=[
            pl.BlockSpec((EB, F_IN), lambda i: (i, 0)),
            pl.BlockSpec((EB, F_IN), lambda i: (i, 0)),
            pl.BlockSpec((EB, ED), lambda i: (i, 0)),
            pl.BlockSpec((ED, F_IN), lambda i: (0, 0)),
            pl.BlockSpec((1, F_IN), lambda i: (0, 0)),
            pl.BlockSpec((3 * F_IN, TOWERS * F_IN), lambda i: (0, 0)),
            pl.BlockSpec((1, TOWERS * F_IN), lambda i: (0, 0)),
            pl.BlockSpec((TOWERS, F_IN, F_IN), lambda i: (0, 0, 0)),
            pl.BlockSpec((1, TOWERS * F_IN), lambda i: (0, 0)),
        ],
        out_specs=pl.BlockSpec((EB, TOWERS * F_IN), lambda i: (i, 0)),
        out_shape=jax.ShapeDtypeStruct((E_PAD, TOWERS * F_IN), jnp.float32),
    )(xi, xj, ea, wee, bee, w0, b0, w1, b1)


def _node_body(x_ref, sum_ref, sq_ref, mx_ref, rc_ref, he_ref,
               w0_ref, b0_ref, w1_ref, b1_ref, wl_ref, bl_ref,
               wih_ref, bih_ref, whh_ref, bhh_ref, out_ref):
    x = x_ref[...]
    rc = rc_ref[...]
    he = he_ref[...]
    mean = sum_ref[...] * rc
    m2 = sq_ref[...] * rc
    var = m2 - mean * mean
    std = jnp.sqrt(jnp.maximum(var, 0.0) + 1e-5)
    mx = jnp.where(he > 0.0, mx_ref[...], 0.0)
    outs = []
    for t in range(TOWERS):
        s = t * F_IN
        hin = jnp.concatenate(
            [x, mean[:, s:s + F_IN], mx[:, s:s + F_IN], std[:, s:s + F_IN]],
            axis=1)
        o = jnp.maximum(hin @ w0_ref[t] + b0_ref[:, t * F_OUT:(t + 1) * F_OUT],
                        0.0)
        outs.append(o @ w1_ref[t] + b1_ref[:, t * F_OUT:(t + 1) * F_OUT])
    mcat = jnp.concatenate(outs, axis=1)
    m = mcat @ wl_ref[...] + bl_ref[...]
    gi = m @ wih_ref[...] + bih_ref[...]
    gh = x @ whh_ref[...] + bhh_ref[...]
    r = jax.nn.sigmoid(gi[:, :H] + gh[:, :H])
    z = jax.nn.sigmoid(gi[:, H:2 * H] + gh[:, H:2 * H])
    n = jnp.tanh(gi[:, 2 * H:] + r * gh[:, 2 * H:])
    out_ref[...] = (1.0 - z) * n + z * x


def _node_call(x, ssum, ssq, smx, rc, he, w0, b0, w1, b1, wl, bl,
               wih, bih, whh, bhh):
    grid = (N // NB,)
    A = TOWERS * F_IN
    return pl.pallas_call(
        _node_body,
        grid=grid,
        in_specs=[
            pl.BlockSpec((NB, F_IN), lambda i: (i, 0)),
            pl.BlockSpec((NB, A), lambda i: (i, 0)),
            pl.BlockSpec((NB, A), lambda i: (i, 0)),
            pl.BlockSpec((NB, A), lambda i: (i, 0)),
            pl.BlockSpec((NB, 1), lambda i: (i, 0)),
            pl.BlockSpec((NB, 1), lambda i: (i, 0)),
            pl.BlockSpec((TOWERS, 4 * F_IN, F_OUT), lambda i: (0, 0, 0)),
            pl.BlockSpec((1, TOWERS * F_OUT), lambda i: (0, 0)),
            pl.BlockSpec((TOWERS, F_OUT, F_OUT), lambda i: (0, 0, 0)),
            pl.BlockSpec((1, TOWERS * F_OUT), lambda i: (0, 0)),
            pl.BlockSpec((H, H), lambda i: (0, 0)),
            pl.BlockSpec((1, H), lambda i: (0, 0)),
            pl.BlockSpec((H, 3 * H), lambda i: (0, 0)),
            pl.BlockSpec((1, 3 * H), lambda i: (0, 0)),
            pl.BlockSpec((H, 3 * H), lambda i: (0, 0)),
            pl.BlockSpec((1, 3 * H), lambda i: (0, 0)),
        ],
        out_specs=pl.BlockSpec((NB, H), lambda i: (i, 0)),
        out_shape=jax.ShapeDtypeStruct((N, H), jnp.float32),
    )(x, ssum, ssq, smx, rc, he, w0, b0, w1, b1, wl, bl, wih, bih, whh, bhh)


def _edge_body(u_ref, v_ref, ea_ref,
               au0_ref, cu0_ref, au1_ref, cu1_ref,
               av0_ref, cv0_ref, av1_ref, cv1_ref,
               f0_ref, g0_ref, f1_ref, g1_ref,
               e0_ref, d0_ref, e1_ref, d1_ref, out_ref):
    fu = jnp.maximum(u_ref[...] @ au0_ref[...] + cu0_ref[...], 0.0)
    fu = jnp.maximum(fu @ au1_ref[...] + cu1_ref[...], 0.0)
    fv = jnp.maximum(v_ref[...] @ av0_ref[...] + cv0_ref[...], 0.0)
    fv = jnp.maximum(fv @ av1_ref[...] + cv1_ref[...], 0.0)
    p = fu * fv
    m1 = jnp.maximum(p @ f0_ref[...] + g0_ref[...], 0.0)
    m_ed = jnp.maximum(m1 @ f1_ref[...] + g1_ref[...], 0.0)
    cat = jnp.concatenate([m_ed, ea_ref[...]], axis=1)
    h = jnp.maximum(cat @ e0_ref[...] + d0_ref[...], 0.0)
    out_ref[...] = h @ e1_ref[...] + d1_ref[...]


def _edge_call(u, v, ea, weights):
    grid = (E // EB,)
    specs = [
        pl.BlockSpec((EB, H), lambda i: (i, 0)),
        pl.BlockSpec((EB, H), lambda i: (i, 0)),
        pl.BlockSpec((EB, ED), lambda i: (i, 0)),
    ]
    for w in weights:
        specs.append(pl.BlockSpec(w.shape, lambda i, _r=len(w.shape): (0,) * _r))
    return pl.pallas_call(
        _edge_body,
        grid=grid,
        in_specs=specs,
        out_specs=pl.BlockSpec((EB, ED), lambda i: (i, 0)),
        out_shape=jax.ShapeDtypeStruct((E, ED), jnp.float32),
    )(u, v, ea, *weights)


def _prep_params(params):
    prepped = {"conv": []}
    for p in params["conv"]:
        q = {}
        q["wee"] = p["ee"][0].T
        q["bee"] = p["ee"][1][None, :]
        q["w0"] = jnp.concatenate([p["pre"][t][0][0].T for t in range(TOWERS)],
                                  axis=1)
        q["b0"] = jnp.concatenate([p["pre"][t][0][1] for t in range(TOWERS)]
                                  )[None, :]
        q["w1"] = jnp.stack([p["pre"][t][1][0].T for t in range(TOWERS)])
        q["b1"] = jnp.concatenate([p["pre"][t][1][1] for t in range(TOWERS)]
                                  )[None, :]
        q["pw0"] = jnp.stack([p["post"][t][0][0].T for t in range(TOWERS)])
        q["pb0"] = jnp.concatenate([p["post"][t][0][1] for t in range(TOWERS)]
                                   )[None, :]
        q["pw1"] = jnp.stack([p["post"][t][1][0].T for t in range(TOWERS)])
        q["pb1"] = jnp.concatenate([p["post"][t][1][1] for t in range(TOWERS)]
                                   )[None, :]
        q["wl"] = p["lin"][0].T
        q["bl"] = p["lin"][1][None, :]
        prepped["conv"].append(q)
    prepped["wih"] = params["gru_Wih"].T
    prepped["bih"] = params["gru_bih"][None, :]
    prepped["whh"] = params["gru_Whh"].T
    prepped["bhh"] = params["gru_bhh"][None, :]
    ew = []
    for key in ("fc_u", "fc_v", "fc", "eu"):
        for (W, b) in params[key]:
            ew.append(W.T)
            ew.append(b[None, :])
    prepped["edge_w"] = ew
    return prepped


NW = 32
GW = 128
CH = 32
NPT = 320
NPAD = NW * NPT
E_PAD = E + CH

_MESH = plsc.VectorSubcoreMesh(core_axis_name="c", subcore_axis_name="s")


def _sc_gather_pair(table, idx_a, idx_b):
    D = table.shape[1]
    PW = 100
    NWIN = 2

    @functools.partial(
        pl.kernel,
        out_type=[jax.ShapeDtypeStruct((E, D), jnp.float32),
                  jax.ShapeDtypeStruct((E, D), jnp.float32)],
        mesh=_MESH,
        scratch_types=[pltpu.SemaphoreType.DMA] * 4,
    )
    def k(tab_hbm, ia_hbm, ib_hbm, oa_hbm, ob_hbm, s0, s1, s2, s3):
        def body(ia_vmem, ib_vmem, oa_vmem, ob_vmem):
            cs = []
            for wdx in range(NWIN):
                cs.append(pltpu.async_copy(
                    tab_hbm.at[ia_vmem.at[wdx]],
                    oa_vmem.at[pl.ds(wdx * PW, PW)], (s0, s1)[wdx]))
                cs.append(pltpu.async_copy(
                    tab_hbm.at[ib_vmem.at[wdx]],
                    ob_vmem.at[pl.ds(wdx * PW, PW)], (s2, s3)[wdx]))
            for c in cs:
                c.wait()

        pltpu.emit_pipeline(
            body,
            grid=(E // (NWIN * PW),),
            in_specs=[
                pl.BlockSpec((NWIN, PW), lambda i: (i, 0)),
                pl.BlockSpec((NWIN, PW), lambda i: (i, 0)),
            ],
            out_specs=[
                pl.BlockSpec((NWIN * PW, D), lambda i: (i, 0)),
                pl.BlockSpec((NWIN * PW, D), lambda i: (i, 0)),
            ],
            core_axis_name=("c", "s"),
            dimension_semantics=(pltpu.PARALLEL,),
        )(ia_hbm, ib_hbm, oa_hbm, ob_hbm)

    return k(table, idx_a.reshape(E // PW, PW), idx_b.reshape(E // PW, PW))


def _sc_gather_one(table, idx):
    D = table.shape[1]

    @functools.partial(
        pl.kernel,
        out_type=jax.ShapeDtypeStruct((E, D), jnp.float32),
        mesh=_MESH,
        compiler_params=pltpu.CompilerParams(use_tc_tiling_on_sc=False),
    )
    def k(tab_hbm, i_hbm, o_hbm):
        def body(i_vmem, o_vmem):
            pltpu.sync_copy(tab_hbm.at[i_vmem.at[0]], o_vmem)

        pltpu.emit_pipeline(
            body,
            grid=(E // GW,),
            in_specs=[pl.BlockSpec((1, GW), lambda i: (0, i))],
            out_specs=[pl.BlockSpec((GW, D), lambda i: (i, 0))],
            core_axis_name=("c", "s"),
            dimension_semantics=(pltpu.PARALLEL,),
        )(i_hbm, o_hbm)

    return k(table, idx)


def _sc_segreduce(msg_padded, off_padded):
    A = TOWERS * F_IN
    G = 8

    @functools.partial(
        pl.kernel,
        out_type=[jax.ShapeDtypeStruct((N, A), jnp.float32)] * 3,
        mesh=_MESH,
        scratch_types=[
            pltpu.VMEM((NPT + 16,), jnp.int32),
            pltpu.VMEM((CH, A), jnp.float32),
            pltpu.VMEM((G, A), jnp.float32),
            pltpu.VMEM((G, A), jnp.float32),
            pltpu.VMEM((G, A), jnp.float32),
        ],
    )
    def k(msg_hbm, off_hbm, sum_hbm, sq_hbm, mx_hbm,
          off_v, buf, st_sum, st_sq, st_mx):
        wid = lax.axis_index("s") * 2 + lax.axis_index("c")
        n0 = pl.multiple_of(wid * NPT, 8)
        pltpu.sync_copy(off_hbm.at[pl.ds(n0, NPT + 16)], off_v)
        nn = jnp.minimum(NPT, N - n0)
        ng = nn // G

        @pl.loop(0, ng)
        def g_body(g):
            @pl.loop(0, G)
            def n_body(kk):
                i = g * G + kk
                ovec = off_v[pl.ds(i, 16)]
                s = ovec[0]
                e = ovec[1]
                zero = jnp.zeros((16,), jnp.float32)
                ninf = jnp.full((16,), -3.4e38, jnp.float32)
                for f in range(A // 16):
                    st_sum[kk, pl.ds(f * 16, 16)] = zero
                    st_sq[kk, pl.ds(f * 16, 16)] = zero
                    st_mx[kk, pl.ds(f * 16, 16)] = ninf
                s_al = (s // 8) * 8
                nch = (e - s_al + (CH - 1)) // CH

                @pl.loop(0, nch)
                def c_body(c, _kk=kk, _s=s, _e=e, _s_al=s_al):
                    base = pl.multiple_of(_s_al + c * CH, 8)
                    pltpu.sync_copy(msg_hbm.at[pl.ds(base, CH)], buf)
                    lo = jnp.maximum(_s - base, 0)
                    valid = jnp.minimum(CH, _e - base)
                    for fq in range(4):
                        def e_body(j, carry, _fq=fq):
                            acc = list(carry)
                            for t in range(8):
                                v = buf[j, pl.ds(_fq * 128 + t * 16, 16)]
                                acc[t] = acc[t] + v
                                acc[8 + t] = acc[8 + t] + v * v
                                acc[16 + t] = jnp.maximum(acc[16 + t], v)
                            return tuple(acc)

                        init = tuple(
                            [st_sum[_kk, pl.ds(fq * 128 + t * 16, 16)]
                             for t in range(8)]
                            + [st_sq[_kk, pl.ds(fq * 128 + t * 16, 16)]
                               for t in range(8)]
                            + [st_mx[_kk, pl.ds(fq * 128 + t * 16, 16)]
                               for t in range(8)])
                        res = lax.fori_loop(lo, valid, e_body, init)
                        for t in range(8):
                            st_sum[_kk, pl.ds(fq * 128 + t * 16, 16)] = res[t]
                            st_sq[_kk, pl.ds(fq * 128 + t * 16, 16)] = res[8 + t]
                            st_mx[_kk, pl.ds(fq * 128 + t * 16, 16)] = res[16 + t]

            row0 = pl.multiple_of(n0 + g * G, 8)
            pltpu.sync_copy(st_sum, sum_hbm.at[pl.ds(row0, G)])
            pltpu.sync_copy(st_sq, sq_hbm.at[pl.ds(row0, G)])
            pltpu.sync_copy(st_mx, mx_hbm.at[pl.ds(row0, G)])

    return k(msg_padded, off_padded)


def kernel(x, edge_index, edge_attr, params):
    src = edge_index[0]
    dst = edge_index[1]
    p = _prep_params(params)

    eid = jnp.arange(E, dtype=jnp.int32)
    sdst, ssrc, perm = lax.sort((dst, src, eid), num_keys=1)
    off = jnp.searchsorted(sdst, jnp.arange(N + 1, dtype=jnp.int32)
                           ).astype(jnp.int32)
    off_padded = jnp.concatenate(
        [off, jnp.full((NPAD + 16 - (N + 1),), E, jnp.int32)])
    cnt = (off[1:] - off[:-1]).astype(jnp.float32)
    rc = (1.0 / jnp.maximum(cnt, 1.0))[:, None]
    he = (cnt > 0).astype(jnp.float32)[:, None]
    invp = jnp.zeros((E,), jnp.int32).at[perm].set(eid)

    sdst2d = sdst.reshape(1, E)
    ssrc2d = ssrc.reshape(1, E)
    ea = _sc_gather_one(edge_attr, perm.reshape(1, E))
    out = x
    u = v = None
    for i in range(LAYERS):
        q = p["conv"][i]
        if i == 0:
            xi, xj = _sc_gather_pair(out, sdst2d, ssrc2d)
        else:
            xi, xj = v, u
        msg = _msg_call(xi, xj, ea, q["wee"], q["bee"], q["w0"], q["b0"],
                        q["w1"], q["b1"])
        ssum, ssq, smx = _sc_segreduce(msg, off_padded)
        out = _node_call(out, ssum, ssq, smx, rc, he,
                         q["pw0"], q["pb0"], q["pw1"], q["pb1"],
                         q["wl"], q["bl"],
                         p["wih"], p["bih"], p["whh"], p["bhh"])
        u, v = _sc_gather_pair(out, ssrc2d, sdst2d)
        ea = _edge_call(u, v, ea, p["edge_w"])
    ea = _sc_gather_one(ea, invp.reshape(1, E))
    return (out, ea, out)
